# baseline (speedup 1.0000x reference)
import jax
import jax.numpy as jnp
from jax.experimental import pallas as pl
from jax.experimental.pallas import tpu as pltpu

N_CLASSES = 1000
N_CTX = 8
NAME_LEN = 77
OUT_LEN = N_CTX + NAME_LEN
CTX_DIM = 512

NCHUNK = 4
CHUNK = N_CLASSES // NCHUNK


def _dma_body(ctx_ref, name_ref, out_ref, ctx_sems, name_sems):
    copies = []
    for k in range(NCHUNK):
        lo = k * CHUNK
        copies.append(
            pltpu.make_async_copy(
                ctx_ref.at[pl.ds(lo, CHUNK)],
                out_ref.at[pl.ds(lo, CHUNK), pl.ds(0, N_CTX)],
                ctx_sems.at[k],
            )
        )
        copies.append(
            pltpu.make_async_copy(
                name_ref.at[pl.ds(lo, CHUNK)],
                out_ref.at[pl.ds(lo, CHUNK), pl.ds(N_CTX, NAME_LEN)],
                name_sems.at[k],
            )
        )
    for c in copies:
        c.start()
    for c in copies:
        c.wait()


def kernel(ctx, name_embeds):
    return pl.pallas_call(
        _dma_body,
        in_specs=[
            pl.BlockSpec(memory_space=pl.ANY),
            pl.BlockSpec(memory_space=pl.ANY),
        ],
        out_specs=pl.BlockSpec(memory_space=pl.ANY),
        out_shape=jax.ShapeDtypeStruct((N_CLASSES, OUT_LEN, CTX_DIM), jnp.float32),
        scratch_shapes=[
            pltpu.SemaphoreType.DMA((NCHUNK,)),
            pltpu.SemaphoreType.DMA((NCHUNK,)),
        ],
    )(ctx, name_embeds)


# speedup vs baseline: 16.2439x; 16.2439x over previous
import jax
import jax.numpy as jnp
from jax.experimental import pallas as pl

N_CLASSES = 1000
N_CTX = 8
NAME_LEN = 77
OUT_LEN = N_CTX + NAME_LEN
CTX_DIM = 512

BLOCK_C = 50


def _concat_body(ctx_ref, name_ref, out_ref):
    out_ref[:, 0:N_CTX, :] = ctx_ref[...]
    out_ref[:, N_CTX:OUT_LEN, :] = name_ref[...]


def kernel(ctx, name_embeds):
    grid = (N_CLASSES // BLOCK_C,)
    return pl.pallas_call(
        _concat_body,
        grid=grid,
        in_specs=[
            pl.BlockSpec((BLOCK_C, N_CTX, CTX_DIM), lambda i: (i, 0, 0)),
            pl.BlockSpec((BLOCK_C, NAME_LEN, CTX_DIM), lambda i: (i, 0, 0)),
        ],
        out_specs=pl.BlockSpec((BLOCK_C, OUT_LEN, CTX_DIM), lambda i: (i, 0, 0)),
        out_shape=jax.ShapeDtypeStruct((N_CLASSES, OUT_LEN, CTX_DIM), jnp.float32),
    )(ctx, name_embeds)
